# Initial kernel scaffold; baseline (speedup 1.0000x reference)
import jax
import jax.numpy as jnp
from jax.experimental import pallas as pl


def kernel(x, edge_index, Wl, bl, Wr, br, att, bias, Wres, gamma, beta):
    raise NotImplementedError("write your pallas kernel here")


# speedup vs baseline: 1.0192x; 1.0192x over previous
import jax
import jax.numpy as jnp
from jax.experimental import pallas as pl
from jax.experimental.pallas import tpu as pltpu


def _ln_body(acc_ref, den_ref, bias_ref, res_ref, gamma_ref, beta_ref, out_ref):
    acc = acc_ref[...]
    den = den_ref[...]
    heads = den.shape[-1]
    c = acc.shape[-1] // heads
    den_b = jnp.repeat(den, c, axis=-1)
    out = acc / den_b + bias_ref[...] + res_ref[...]
    mu = jnp.mean(out, axis=-1, keepdims=True)
    var = jnp.mean((out - mu) ** 2, axis=-1, keepdims=True)
    out_ref[...] = gamma_ref[...] * (out - mu) * jax.lax.rsqrt(var + 1e-5) + beta_ref[...]


def _finalize(acc, den, bias, res, gamma, beta):
    n, hid = acc.shape
    blk = 1000
    return pl.pallas_call(
        _ln_body,
        grid=(n // blk,),
        in_specs=[
            pl.BlockSpec((blk, hid), lambda i: (i, 0)),
            pl.BlockSpec((blk, den.shape[1]), lambda i: (i, 0)),
            pl.BlockSpec((1, hid), lambda i: (0, 0)),
            pl.BlockSpec((blk, hid), lambda i: (i, 0)),
            pl.BlockSpec((1, hid), lambda i: (0, 0)),
            pl.BlockSpec((1, hid), lambda i: (0, 0)),
        ],
        out_specs=pl.BlockSpec((blk, hid), lambda i: (i, 0)),
        out_shape=jax.ShapeDtypeStruct((n, hid), jnp.float32),
    )(acc, den, bias.reshape(1, hid), res, gamma.reshape(1, hid), beta.reshape(1, hid))


def kernel(x, edge_index, Wl, bl, Wr, br, att, bias, Wres, gamma, beta):
    n = x.shape[0]
    heads, c = att.shape
    loops = jnp.arange(n, dtype=edge_index.dtype)
    src = jnp.concatenate([edge_index[0], loops])
    dst = jnp.concatenate([edge_index[1], loops])
    xl = (x @ Wl + bl).reshape(n, heads, c)
    xr = (x @ Wr + br).reshape(n, heads, c)
    e = jax.nn.leaky_relu(xl[src] + xr[dst], negative_slope=0.2)
    logits = (e * att[None, :, :]).sum(-1)
    m = jax.ops.segment_max(logits, dst, num_segments=n)
    alpha = jnp.exp(logits - m[dst])
    den = jax.ops.segment_sum(alpha, dst, num_segments=n)
    acc = jax.ops.segment_sum(xl[src] * alpha[..., None], dst, num_segments=n)
    acc = acc.reshape(n, heads * c)
    return _finalize(acc, den, bias, x @ Wres, gamma, beta)


# speedup vs baseline: 2.9229x; 2.8678x over previous
import functools

import jax
import jax.numpy as jnp
from jax import lax
from jax.experimental import pallas as pl
from jax.experimental.pallas import tpu as pltpu
from jax.experimental.pallas import tpu_sc as plsc

N = 10000
NROW = N + 1
RPT = 632
NTAB = 16 * RPT
HALF = 128
BE = 64
E2 = 320000 + N
TOTAL = ((E2 + 16 * (BE - 1)) // BE) * BE


def _mm_body(x_ref, w_ref, b_ref, o_ref):
    o_ref[...] = (
        jnp.dot(x_ref[...], w_ref[...], preferred_element_type=jnp.float32)
        + b_ref[...]
    )


def _fused_matmul(x, wcat, bcat):
    n, k = x.shape
    m = wcat.shape[1]
    blk = 1000
    return pl.pallas_call(
        _mm_body,
        grid=(n // blk,),
        in_specs=[
            pl.BlockSpec((blk, k), lambda i: (i, 0)),
            pl.BlockSpec((k, m), lambda i: (0, 0)),
            pl.BlockSpec((1, m), lambda i: (0, 0)),
        ],
        out_specs=pl.BlockSpec((blk, m), lambda i: (i, 0)),
        out_shape=jax.ShapeDtypeStruct((n, m), jnp.float32),
    )(x, wcat, bcat.reshape(1, m))


def _ln_body(acc_ref, den_ref, res_ref, gamma_ref, beta_ref, out_ref):
    halves = []
    for h in range(2):
        den = den_ref[h][:, :4]
        den_b = jnp.repeat(den, 32, axis=-1)
        halves.append(acc_ref[h] / den_b)
    out = jnp.concatenate(halves, axis=-1) + res_ref[...]
    mu = jnp.mean(out, axis=-1, keepdims=True)
    var = jnp.mean((out - mu) ** 2, axis=-1, keepdims=True)
    out_ref[...] = gamma_ref[...] * (out - mu) * lax.rsqrt(var + 1e-5) + beta_ref[...]


def _finalize(acc, den, res, gamma, beta):
    hid = res.shape[1]
    blk = 1000
    return pl.pallas_call(
        _ln_body,
        grid=(N // blk,),
        in_specs=[
            pl.BlockSpec((2, blk, HALF), lambda i: (0, i, 0)),
            pl.BlockSpec((2, blk, 4), lambda i: (0, i, 0)),
            pl.BlockSpec((blk, hid), lambda i: (i, 0)),
            pl.BlockSpec((1, hid), lambda i: (0, 0)),
            pl.BlockSpec((1, hid), lambda i: (0, 0)),
        ],
        out_specs=pl.BlockSpec((blk, hid), lambda i: (i, 0)),
        out_shape=jax.ShapeDtypeStruct((N, hid), jnp.float32),
    )(acc, den, res, gamma.reshape(1, hid), beta.reshape(1, hid))


def _make_sc_kernel():
    mesh = plsc.VectorSubcoreMesh(core_axis_name="c", subcore_axis_name="s")

    @functools.partial(
        pl.kernel,
        mesh=mesh,
        compiler_params=pltpu.CompilerParams(needs_layout_passes=False),
        out_type=[
            jax.ShapeDtypeStruct((2, NTAB, HALF), jnp.float32),
            jax.ShapeDtypeStruct((2, 16, 4, RPT), jnp.float32),
        ],
        scratch_types=[
            pltpu.VMEM((BE,), jnp.int32),
            pltpu.VMEM((BE,), jnp.int32),
            pltpu.VMEM((BE,), jnp.int32),
            pltpu.VMEM((BE, HALF), jnp.float32),
            pltpu.VMEM((BE, HALF), jnp.float32),
            pltpu.VMEM((4, BE), jnp.float32),
            pltpu.VMEM((128,), jnp.float32),
            pltpu.VMEM((RPT, HALF), jnp.float32),
            pltpu.VMEM((4, RPT), jnp.float32),
            pltpu.SemaphoreType.DMA,
            pltpu.SemaphoreType.DMA,
        ],
    )
    def sc_kernel(xl_hbm, xr_hbm, srcg_hbm, dstg_hbm, dstl_hbm, att_hbm,
                  starts_hbm, nch_hbm,
                  acc_out, den_out, s_v, d_v, dg_v, srows, drows, arows, att_v,
                  acc_l, den_l, sem1, sem2):
        core = lax.axis_index("c")
        sub = lax.axis_index("s")
        lanes = lax.broadcasted_iota(jnp.int32, (16,), 0)
        p_row = lax.shift_right_logical(lanes, 2)
        p_head = jnp.bitwise_and(lanes, 3)
        p_col = p_head * 32
        th = jnp.full((16,), core * NROW + N, jnp.int32)

        pltpu.sync_copy(att_hbm.at[pl.ds(core * 128, 128)], att_v)

        def _zero(r, _):
            for v in range(8):
                acc_l[r, pl.ds(v * 16, 16)] = jnp.zeros((16,), jnp.float32)
            return 0
        lax.fori_loop(0, RPT, _zero, 0)

        def _zero_den(r, _):
            for h in range(4):
                den_l[h, pl.ds(r * 16, 16)] = jnp.zeros((16,), jnp.float32)
            return 0
        lax.fori_loop(0, RPT // 16, _zero_den, 0)

        pltpu.sync_copy(starts_hbm, s_v.at[pl.ds(0, 16)])
        pltpu.sync_copy(nch_hbm, s_v.at[pl.ds(16, 16)])
        me = lanes == jnp.full((16,), sub, jnp.int32)
        zero16 = jnp.zeros((16,), jnp.int32)
        start = lax.reduce_max(jnp.where(me, s_v[pl.ds(0, 16)], zero16), (0,))
        start = pl.multiple_of(start, BE)
        nch = lax.reduce_max(jnp.where(me, s_v[pl.ds(16, 16)], zero16), (0,))

        def _chunk(ci, _):
            base = start + ci * BE
            gbase = core * TOTAL + base
            pltpu.sync_copy(srcg_hbm.at[pl.ds(gbase, BE)], s_v)
            pltpu.sync_copy(dstg_hbm.at[pl.ds(gbase, BE)], dg_v)
            pltpu.sync_copy(dstl_hbm.at[pl.ds(base, BE)], d_v)
            h1 = pltpu.async_copy(xl_hbm.at[s_v], srows, sem1)
            h2 = pltpu.async_copy(xr_hbm.at[dg_v], drows, sem2)
            h1.wait()
            h2.wait()

            def _group(g, _):
                rows16 = p_row + jnp.full((16,), g * 4, jnp.int32)

                def _chan(j, acc):
                    cols = p_col + jnp.full((16,), j, jnp.int32)
                    s = plsc.load_gather(srows, [rows16, cols])
                    d = plsc.load_gather(drows, [rows16, cols])
                    z = s + d
                    z = jnp.maximum(z, z * 0.2)
                    a = plsc.load_gather(att_v, [cols])
                    return acc + z * a

                logit = lax.fori_loop(0, 32, _chan, jnp.zeros((16,), jnp.float32))
                alpha = jnp.exp(logit)
                sv = plsc.load_gather(s_v, [rows16])
                alpha = jnp.where(sv >= th, jnp.zeros((16,), jnp.float32), alpha)
                plsc.store_scatter(arows, [p_head, rows16], alpha)
                return 0

            lax.fori_loop(0, BE * 4 // 16, _group, 0)

            for g2 in range(BE // 16):
                dlv = d_v[pl.ds(g2 * 16, 16)]
                erows = lanes + jnp.full((16,), g2 * 16, jnp.int32)
                for h in range(4):
                    hsp = jnp.full((16,), h, jnp.int32)
                    av = plsc.load_gather(arows, [hsp, erows])
                    plsc.addupdate_scatter(den_l, [hsp, dlv], av)

                    def _bd(j, _, av=av, dlv=dlv, erows=erows, h=h):
                        col = jnp.full((16,), h * 32, jnp.int32) + jnp.full(
                            (16,), j, jnp.int32)
                        vals = plsc.load_gather(srows, [erows, col]) * av
                        plsc.addupdate_scatter(acc_l, [dlv, col], vals)
                        return 0

                    lax.fori_loop(0, 32, _bd, 0)
            return 0

        lax.fori_loop(0, nch, _chunk, 0)

        pltpu.sync_copy(acc_l, acc_out.at[core, pl.ds(sub * RPT, RPT)])
        pltpu.sync_copy(den_l, den_out.at[core, sub])

    return sc_kernel


def kernel(x, edge_index, Wl, bl, Wr, br, att, bias, Wres, gamma, beta):
    n, in_dim = x.shape
    heads, c = att.shape
    hid = heads * c

    wcat = jnp.concatenate([Wl, Wr, Wres], axis=1)
    bcat = jnp.concatenate([bl, br, bias], axis=0)
    xcat = _fused_matmul(x, wcat, bcat)
    xl = xcat[:, :hid]
    xr = xcat[:, hid:2 * hid]
    res = xcat[:, 2 * hid:]

    def to_table(a):
        a = a.reshape(n, 2, HALF).transpose(1, 0, 2)
        a = jnp.concatenate([a, jnp.zeros((2, 1, HALF), jnp.float32)], axis=1)
        return a.reshape(2 * NROW, HALF)

    xl_t = to_table(xl)
    xr_t = to_table(xr)

    loops = jnp.arange(n, dtype=jnp.int32)
    src_a = jnp.concatenate([edge_index[0], loops])
    dst_a = jnp.concatenate([edge_index[1], loops])
    order = jnp.argsort(dst_a)
    src_s = src_a[order]
    dst_s = dst_a[order]
    tile_of = dst_s // RPT
    bounds = jnp.searchsorted(dst_s, jnp.arange(17, dtype=jnp.int32) * RPT)
    bounds = bounds.astype(jnp.int32)
    cnt = jnp.diff(bounds)
    pcnt = (((cnt + BE - 1) // BE) * BE).astype(jnp.int32)
    offs = jnp.concatenate([jnp.zeros((1,), jnp.int32),
                            jnp.cumsum(pcnt)[:-1].astype(jnp.int32)])
    pos = offs[tile_of] + jnp.arange(E2, dtype=jnp.int32) - bounds[tile_of]
    srcp = jnp.full((TOTAL,), N, jnp.int32).at[pos].set(src_s)
    dstp = jnp.zeros((TOTAL,), jnp.int32).at[pos].set(dst_s)
    dstl = jnp.zeros((TOTAL,), jnp.int32).at[pos].set(dst_s - tile_of * RPT)
    srcg = jnp.concatenate([srcp, srcp + NROW])
    dstg = jnp.concatenate([dstp, dstp + NROW])

    sc = _make_sc_kernel()
    acc, den4 = sc(
        xl_t, xr_t, srcg, dstg, dstl, att.reshape(-1),
        offs, (pcnt // BE).astype(jnp.int32),
    )
    den = den4.transpose(0, 1, 3, 2).reshape(2, NTAB, 4)
    return _finalize(acc, den, res, gamma, beta)


# speedup vs baseline: 3.0741x; 1.0517x over previous
import functools

import jax
import jax.numpy as jnp
from jax import lax
from jax.experimental import pallas as pl
from jax.experimental.pallas import tpu as pltpu
from jax.experimental.pallas import tpu_sc as plsc

N = 10000
NROW = N + 1
RPT = 632
NTAB = 16 * RPT
HALF = 128
BE = 64
E2 = 320000 + N
TOTAL = ((E2 + 16 * (BE - 1)) // BE) * BE


def _mm_body(x_ref, w_ref, b_ref, o_ref):
    o_ref[...] = (
        jnp.dot(x_ref[...], w_ref[...], preferred_element_type=jnp.float32)
        + b_ref[...]
    )


def _fused_matmul(x, wcat, bcat):
    n, k = x.shape
    m = wcat.shape[1]
    blk = 1000
    return pl.pallas_call(
        _mm_body,
        grid=(n // blk,),
        in_specs=[
            pl.BlockSpec((blk, k), lambda i: (i, 0)),
            pl.BlockSpec((k, m), lambda i: (0, 0)),
            pl.BlockSpec((1, m), lambda i: (0, 0)),
        ],
        out_specs=pl.BlockSpec((blk, m), lambda i: (i, 0)),
        out_shape=jax.ShapeDtypeStruct((n, m), jnp.float32),
    )(x, wcat, bcat.reshape(1, m))


def _ln_body(acc_ref, den_ref, res_ref, gamma_ref, beta_ref, out_ref):
    halves = []
    for h in range(2):
        den = den_ref[h][:, :4]
        den_b = jnp.repeat(den, 32, axis=-1)
        halves.append(acc_ref[h] / den_b)
    out = jnp.concatenate(halves, axis=-1) + res_ref[...]
    mu = jnp.mean(out, axis=-1, keepdims=True)
    var = jnp.mean((out - mu) ** 2, axis=-1, keepdims=True)
    out_ref[...] = gamma_ref[...] * (out - mu) * lax.rsqrt(var + 1e-5) + beta_ref[...]


def _finalize(acc, den, res, gamma, beta):
    hid = res.shape[1]
    blk = 1000
    return pl.pallas_call(
        _ln_body,
        grid=(N // blk,),
        in_specs=[
            pl.BlockSpec((2, blk, HALF), lambda i: (0, i, 0)),
            pl.BlockSpec((2, blk, 4), lambda i: (0, i, 0)),
            pl.BlockSpec((blk, hid), lambda i: (i, 0)),
            pl.BlockSpec((1, hid), lambda i: (0, 0)),
            pl.BlockSpec((1, hid), lambda i: (0, 0)),
        ],
        out_specs=pl.BlockSpec((blk, hid), lambda i: (i, 0)),
        out_shape=jax.ShapeDtypeStruct((N, hid), jnp.float32),
    )(acc, den, res, gamma.reshape(1, hid), beta.reshape(1, hid))


def _make_sc_kernel():
    mesh = plsc.VectorSubcoreMesh(core_axis_name="c", subcore_axis_name="s")

    @functools.partial(
        pl.kernel,
        mesh=mesh,
        compiler_params=pltpu.CompilerParams(needs_layout_passes=False),
        out_type=[
            jax.ShapeDtypeStruct((2, NTAB, HALF), jnp.float32),
            jax.ShapeDtypeStruct((2, 16, 4, RPT), jnp.float32),
        ],
        scratch_types=[
            pltpu.VMEM((BE,), jnp.int32),
            pltpu.VMEM((BE,), jnp.int32),
            pltpu.VMEM((BE,), jnp.int32),
            pltpu.VMEM((BE, HALF), jnp.float32),
            pltpu.VMEM((BE, HALF), jnp.float32),
            pltpu.VMEM((4, BE), jnp.float32),
            pltpu.VMEM((128,), jnp.float32),
            pltpu.VMEM((RPT, HALF), jnp.float32),
            pltpu.VMEM((4, RPT), jnp.float32),
            pltpu.SemaphoreType.DMA,
            pltpu.SemaphoreType.DMA,
        ],
    )
    def sc_kernel(xl_hbm, xr_hbm, srcg_hbm, dstg_hbm, dstl_hbm, att_hbm,
                  starts_hbm, nch_hbm,
                  acc_out, den_out, s_v, d_v, dg_v, srows, drows, arows, att_v,
                  acc_l, den_l, sem1, sem2):
        core = lax.axis_index("c")
        sub = lax.axis_index("s")
        lanes = lax.broadcasted_iota(jnp.int32, (16,), 0)
        p_row = lax.shift_right_logical(lanes, 2)
        p_head = jnp.bitwise_and(lanes, 3)
        p_col = p_head * 32
        th = jnp.full((16,), core * NROW + N, jnp.int32)

        pltpu.sync_copy(att_hbm.at[pl.ds(core * 128, 128)], att_v)

        def _zero(r, _):
            for v in range(8):
                acc_l[r, pl.ds(v * 16, 16)] = jnp.zeros((16,), jnp.float32)
            return 0
        lax.fori_loop(0, RPT, _zero, 0)

        def _zero_den(r, _):
            for h in range(4):
                den_l[h, pl.ds(r * 16, 16)] = jnp.zeros((16,), jnp.float32)
            return 0
        lax.fori_loop(0, RPT // 16, _zero_den, 0)

        pltpu.sync_copy(starts_hbm, s_v.at[pl.ds(0, 16)])
        pltpu.sync_copy(nch_hbm, s_v.at[pl.ds(16, 16)])
        me = lanes == jnp.full((16,), sub, jnp.int32)
        zero16 = jnp.zeros((16,), jnp.int32)
        start = lax.reduce_max(jnp.where(me, s_v[pl.ds(0, 16)], zero16), (0,))
        start = pl.multiple_of(start, BE)
        nch = lax.reduce_max(jnp.where(me, s_v[pl.ds(16, 16)], zero16), (0,))

        def _chunk(ci, _):
            base = start + ci * BE
            gbase = core * TOTAL + base
            pltpu.sync_copy(srcg_hbm.at[pl.ds(gbase, BE)], s_v)
            pltpu.sync_copy(dstg_hbm.at[pl.ds(gbase, BE)], dg_v)
            pltpu.sync_copy(dstl_hbm.at[pl.ds(base, BE)], d_v)
            h1 = pltpu.async_copy(xl_hbm.at[s_v], srows, sem1)
            h2 = pltpu.async_copy(xr_hbm.at[dg_v], drows, sem2)
            h1.wait()
            h2.wait()

            def _group(g, _):
                rows16 = p_row + jnp.full((16,), g * 4, jnp.int32)

                def _chan(j4, acc):
                    for u in range(8):
                        cols = p_col + jnp.full((16,), j4 * 8 + u, jnp.int32)
                        s = plsc.load_gather(srows, [rows16, cols])
                        d = plsc.load_gather(drows, [rows16, cols])
                        z = s + d
                        z = jnp.maximum(z, z * 0.2)
                        a = plsc.load_gather(att_v, [cols])
                        acc = acc + z * a
                    return acc

                logit = lax.fori_loop(0, 4, _chan, jnp.zeros((16,), jnp.float32))
                alpha = jnp.exp(logit)
                sv = plsc.load_gather(s_v, [rows16])
                alpha = jnp.where(sv >= th, jnp.zeros((16,), jnp.float32), alpha)
                plsc.store_scatter(arows, [p_head, rows16], alpha)
                return 0

            lax.fori_loop(0, BE * 4 // 16, _group, 0)

            for g2 in range(BE // 16):
                dlv = d_v[pl.ds(g2 * 16, 16)]
                erows = lanes + jnp.full((16,), g2 * 16, jnp.int32)
                for h in range(4):
                    hsp = jnp.full((16,), h, jnp.int32)
                    av = plsc.load_gather(arows, [hsp, erows])
                    plsc.addupdate_scatter(den_l, [hsp, dlv], av)

                    def _bd(j4, _, av=av, dlv=dlv, erows=erows, h=h):
                        for u in range(8):
                            col = jnp.full((16,), h * 32 + u, jnp.int32) + jnp.full(
                                (16,), j4 * 8, jnp.int32)
                            vals = plsc.load_gather(srows, [erows, col]) * av
                            plsc.addupdate_scatter(acc_l, [dlv, col], vals)
                        return 0

                    lax.fori_loop(0, 4, _bd, 0)
            return 0

        lax.fori_loop(0, nch, _chunk, 0)

        pltpu.sync_copy(acc_l, acc_out.at[core, pl.ds(sub * RPT, RPT)])
        pltpu.sync_copy(den_l, den_out.at[core, sub])

    return sc_kernel


def kernel(x, edge_index, Wl, bl, Wr, br, att, bias, Wres, gamma, beta):
    n, in_dim = x.shape
    heads, c = att.shape
    hid = heads * c

    wcat = jnp.concatenate([Wl, Wr, Wres], axis=1)
    bcat = jnp.concatenate([bl, br, bias], axis=0)
    xcat = _fused_matmul(x, wcat, bcat)
    xl = xcat[:, :hid]
    xr = xcat[:, hid:2 * hid]
    res = xcat[:, 2 * hid:]

    def to_table(a):
        a = a.reshape(n, 2, HALF).transpose(1, 0, 2)
        a = jnp.concatenate([a, jnp.zeros((2, 1, HALF), jnp.float32)], axis=1)
        return a.reshape(2 * NROW, HALF)

    xl_t = to_table(xl)
    xr_t = to_table(xr)

    loops = jnp.arange(n, dtype=jnp.int32)
    src_a = jnp.concatenate([edge_index[0], loops])
    dst_a = jnp.concatenate([edge_index[1], loops])
    order = jnp.argsort(dst_a)
    src_s = src_a[order]
    dst_s = dst_a[order]
    tile_of = dst_s // RPT
    bounds = jnp.searchsorted(dst_s, jnp.arange(17, dtype=jnp.int32) * RPT)
    bounds = bounds.astype(jnp.int32)
    cnt = jnp.diff(bounds)
    pcnt = (((cnt + BE - 1) // BE) * BE).astype(jnp.int32)
    offs = jnp.concatenate([jnp.zeros((1,), jnp.int32),
                            jnp.cumsum(pcnt)[:-1].astype(jnp.int32)])
    pos = offs[tile_of] + jnp.arange(E2, dtype=jnp.int32) - bounds[tile_of]
    srcp = jnp.full((TOTAL,), N, jnp.int32).at[pos].set(src_s)
    dstp = jnp.zeros((TOTAL,), jnp.int32).at[pos].set(dst_s)
    dstl = jnp.zeros((TOTAL,), jnp.int32).at[pos].set(dst_s - tile_of * RPT)
    srcg = jnp.concatenate([srcp, srcp + NROW])
    dstg = jnp.concatenate([dstp, dstp + NROW])

    sc = _make_sc_kernel()
    acc, den4 = sc(
        xl_t, xr_t, srcg, dstg, dstl, att.reshape(-1),
        offs, (pcnt // BE).astype(jnp.int32),
    )
    den = den4.transpose(0, 1, 3, 2).reshape(2, NTAB, 4)
    return _finalize(acc, den, res, gamma, beta)
